# baseline (speedup 1.0000x reference)
import functools

import jax
import jax.numpy as jnp
from jax import lax
from jax.experimental import pallas as pl
from jax.experimental.pallas import tpu as pltpu
from jax.experimental.pallas import tpu_sc as plsc

TOP_K = 8
NUM_EXPERTS = 64
N_WORKERS = 32
LANES = 16


def _matmul_kernel(x_ref, w_ref, logits_ref, logits_t_ref):
    x = x_ref[...]
    w = w_ref[...]
    lt = lax.dot_general(
        w, x,
        dimension_numbers=(((1,), (1,)), ((), ())),
        preferred_element_type=jnp.float32,
    )
    logits_t_ref[...] = lt
    logits_ref[...] = lt.T


@functools.partial(jax.jit, static_argnames=("block_t",))
def _matmul(x_flat, weight, block_t=2048):
    n_tokens, hidden = x_flat.shape
    grid = (n_tokens // block_t,)
    return pl.pallas_call(
        _matmul_kernel,
        grid=grid,
        in_specs=[
            pl.BlockSpec((block_t, hidden), lambda i: (i, 0)),
            pl.BlockSpec((NUM_EXPERTS, hidden), lambda i: (0, 0)),
        ],
        out_specs=[
            pl.BlockSpec((block_t, NUM_EXPERTS), lambda i: (i, 0)),
            pl.BlockSpec((NUM_EXPERTS, block_t), lambda i: (0, i)),
        ],
        out_shape=[
            jax.ShapeDtypeStruct((n_tokens, NUM_EXPERTS), jnp.float32),
            jax.ShapeDtypeStruct((NUM_EXPERTS, n_tokens), jnp.float32),
        ],
    )(x_flat, weight)


def _make_sc_topk(n_tokens):
    t_per_w = n_tokens // N_WORKERS
    n_chunks = t_per_w // LANES
    mesh = plsc.VectorSubcoreMesh(core_axis_name="c", subcore_axis_name="s")

    @functools.partial(
        pl.kernel, mesh=mesh,
        out_type=[
            jax.ShapeDtypeStruct((TOP_K, n_tokens), jnp.float32),
            jax.ShapeDtypeStruct((TOP_K, n_tokens), jnp.int32),
        ],
        scratch_types=[
            pltpu.VMEM((NUM_EXPERTS, t_per_w), jnp.float32),
            pltpu.VMEM((TOP_K, t_per_w), jnp.float32),
            pltpu.VMEM((TOP_K, t_per_w), jnp.int32),
        ],
    )
    def sc_topk(logits_t_hbm, topv_hbm, topi_hbm, lbuf, vbuf, ibuf):
        wid = lax.axis_index("s") * 2 + lax.axis_index("c")
        base = wid * t_per_w
        pltpu.sync_copy(logits_t_hbm.at[:, pl.ds(base, t_per_w)], lbuf)

        def chunk_body(c, carry):
            cols = pl.ds(c * LANES, LANES)
            l = [lbuf[e, cols] for e in range(NUM_EXPERTS)]
            m = l[0]
            for e in range(1, NUM_EXPERTS):
                m = jnp.maximum(m, l[e])
            exps = [jnp.exp(l[e] - m) for e in range(NUM_EXPERTS)]
            z = exps[0]
            for e in range(1, NUM_EXPERTS):
                z = z + exps[e]
            scores = [exps[e] / z for e in range(NUM_EXPERTS)]

            work = scores
            vals = []
            idxs = []
            minus1 = jnp.full((LANES,), -1.0, jnp.float32)
            big = jnp.full((LANES,), NUM_EXPERTS, jnp.int32)
            for _ in range(TOP_K):
                mx = work[0]
                for e in range(1, NUM_EXPERTS):
                    mx = jnp.maximum(mx, work[e])
                cand = big
                for e in range(NUM_EXPERTS - 1, -1, -1):
                    idx_e = jnp.full((LANES,), e, jnp.int32)
                    cand = jnp.where(work[e] == mx, idx_e, cand)
                vals.append(mx)
                idxs.append(cand)
                work = [
                    jnp.where(
                        cand == jnp.full((LANES,), e, jnp.int32),
                        minus1, work[e])
                    for e in range(NUM_EXPERTS)
                ]
            tot = vals[0]
            for k in range(1, TOP_K):
                tot = tot + vals[k]
            for k in range(TOP_K):
                vbuf[k, cols] = vals[k] / tot
                ibuf[k, cols] = idxs[k]
            return carry

        lax.fori_loop(0, n_chunks, chunk_body, 0)
        pltpu.sync_copy(vbuf, topv_hbm.at[:, pl.ds(base, t_per_w)])
        pltpu.sync_copy(ibuf, topi_hbm.at[:, pl.ds(base, t_per_w)])

    return sc_topk


def kernel(x, weight):
    hidden = weight.shape[1]
    x_flat = x.reshape(-1, hidden)
    n_tokens = x_flat.shape[0]
    logits, logits_t = _matmul(x_flat, weight)
    topv_t, topi_t = _make_sc_topk(n_tokens)(logits_t)
    return (logits, topv_t.T, topi_t.T)
